# baseline (speedup 1.0000x reference)
import dataclasses
import functools

import jax
import jax.numpy as jnp
import numpy as np
from jax import lax
from jax.experimental import pallas as pl
from jax.experimental.pallas import tpu as pltpu
from jax.experimental.pallas import tpu_sc as plsc

NB_GENES = 300
NB_CHROM = 10
GPC = 30
HIDDEN = 4
SIG = 16
BATCH = 16384
NB_CAT = 3

NUM_CORES = 2
NUM_SUBCORES = 16
NUM_TILES = NUM_CORES * NUM_SUBCORES
LANES = 16

ROWS_PER_TILE = BATCH // NUM_TILES
GROUP = 16
GROUPS = ROWS_PER_TILE // GROUP

XWORDS = NB_GENES // 4
TBL = NB_GENES * NB_CAT
TBL_PAD = 912
EMB_WORDS = TBL * HIDDEN
EBUF_WORDS = TBL_PAD * HIDDEN
RSTRIDE = 920


def _sc_stage(emb_flat, cwb, xqs_in):
    mesh = plsc.VectorSubcoreMesh(core_axis_name="c", subcore_axis_name="s")
    cp = pltpu.CompilerParams()
    if "needs_layout_passes" in pltpu.CompilerParams.__dataclass_fields__:
        cp = dataclasses.replace(cp, needs_layout_passes=False)

    @functools.partial(
        pl.kernel,
        compiler_params=cp,
        out_type=jax.ShapeDtypeStruct((BATCH, NB_CHROM), jnp.float32),
        mesh=mesh,
        scratch_types=[
            pltpu.VMEM((EBUF_WORDS,), jnp.float32),
            pltpu.VMEM((5 * LANES,), jnp.float32),
            pltpu.VMEM((TBL_PAD,), jnp.float32),
            pltpu.VMEM((GROUP, NB_GENES // 4), jnp.int32),
            pltpu.VMEM((GROUP, NB_GENES // 4), jnp.int32),
            pltpu.VMEM((GROUP, NB_GENES // 4), jnp.int32),
            pltpu.VMEM((GROUP, NB_GENES // 4), jnp.int32),
            pltpu.VMEM((GROUP, NB_CHROM), jnp.float32),
        ],
    )
    def body(emb_hbm, cwb_hbm, xh0, xh1, xh2, xh3, out_hbm, ebuf, cwbuf, tbl,
             xq0, xq1, xq2, xq3, obuf):
        xhs = [xh0, xh1, xh2, xh3]
        wid = lax.axis_index("s") * NUM_CORES + lax.axis_index("c")
        base = wid * ROWS_PER_TILE

        pltpu.sync_copy(emb_hbm, ebuf.at[pl.ds(0, EMB_WORDS)])
        pltpu.sync_copy(cwb_hbm, cwbuf)

        lanes = jnp.arange(LANES, dtype=jnp.int32)
        cw0 = cwbuf[pl.ds(0 * LANES, LANES)]
        cw1 = cwbuf[pl.ds(1 * LANES, LANES)]
        cw2 = cwbuf[pl.ds(2 * LANES, LANES)]
        cw3 = cwbuf[pl.ds(3 * LANES, LANES)]
        cb = cwbuf[pl.ds(4 * LANES, LANES)]

        @pl.loop(0, TBL_PAD, step=LANES)
        def _(i):
            idx0 = (i + lanes) * HIDDEN
            v = (plsc.load_gather(ebuf, [idx0]) * cw0
                 + plsc.load_gather(ebuf, [idx0 + 1]) * cw1
                 + plsc.load_gather(ebuf, [idx0 + 2]) * cw2
                 + plsc.load_gather(ebuf, [idx0 + 3]) * cw3
                 + cb)
            tbl[pl.ds(i, LANES)] = v

        QW = NB_GENES // 4
        xqs = [xq0, xq1, xq2, xq3]

        def compute(row0):
            for c in range(NB_CHROM):
                vals = []
                for j in range(GPC):
                    g = c * GPC + j
                    gv = jnp.full((LANES,), g % QW, jnp.int32)
                    xv = plsc.load_gather(xqs[g // QW], [lanes, gv])
                    tv = plsc.load_gather(tbl, [xv + 3 * g])
                    vals.append(tv)
                while len(vals) > 1:
                    vals = [a + b for a, b in zip(vals[::2], vals[1::2])] \
                        + ([vals[-1]] if len(vals) % 2 else [])
                cv = jnp.full((LANES,), c, jnp.int32)
                plsc.store_scatter(obuf, [lanes, cv], vals[0])
            pltpu.sync_copy(obuf, out_hbm.at[pl.ds(row0, GROUP), :])

        @pl.loop(0, GROUPS)
        def _(grp):
            row0 = base + grp * GROUP
            for q in range(4):
                pltpu.sync_copy(xhs[q].at[pl.ds(row0, GROUP), :], xqs[q])
            compute(row0)

    return body(emb_flat, cwb, *xqs_in)


def _tc_tail(conv, lin_w, lin_b, bn_gamma, bn_beta):
    def body(a_ref, w_ref, b_ref, g_ref, bb_ref, o_ref):
        a = jnp.maximum(a_ref[...], 0.0)
        z = jnp.dot(a, w_ref[...],
                    preferred_element_type=jnp.float32)
        z = z + b_ref[...]
        mean = jnp.mean(z, axis=0, keepdims=True)
        zc = z - mean
        var = jnp.mean(zc * zc, axis=0, keepdims=True)
        inv = lax.rsqrt(var + 1e-5)
        o_ref[...] = zc * (inv * g_ref[...]) + bb_ref[...]

    return pl.pallas_call(
        body,
        out_shape=jax.ShapeDtypeStruct((BATCH, SIG), jnp.float32),
    )(conv, lin_w, lin_b, bn_gamma, bn_beta)


def kernel(x, emb, conv_w, conv_b, lin_w, lin_b, bn_gamma, bn_beta):
    emb_flat = emb.reshape(-1)
    cw = jnp.concatenate([conv_w, conv_b.reshape(1)]) * np.float32(1.0 / GPC)
    cwb = jnp.broadcast_to(cw[:, None], (5, LANES)).reshape(-1)
    qw = NB_GENES // 4
    xqs = [x[:, q * qw:(q + 1) * qw] for q in range(4)]
    conv = _sc_stage(emb_flat, cwb, xqs)
    return _tc_tail(conv, lin_w,
                    lin_b.reshape(1, SIG),
                    bn_gamma.reshape(1, SIG),
                    bn_beta.reshape(1, SIG))


# speedup vs baseline: 1.6886x; 1.6886x over previous
import dataclasses
import functools

import jax
import jax.numpy as jnp
import numpy as np
from jax import lax
from jax.experimental import pallas as pl
from jax.experimental.pallas import tpu as pltpu
from jax.experimental.pallas import tpu_sc as plsc

NB_GENES = 300
NB_CHROM = 10
GPC = 30
HIDDEN = 4
SIG = 16
BATCH = 16384
NB_CAT = 3

NUM_CORES = 2
NUM_SUBCORES = 16
NUM_TILES = NUM_CORES * NUM_SUBCORES
LANES = 16

ROWS_PER_TILE = BATCH // NUM_TILES
GROUP = 16
GROUPS = ROWS_PER_TILE // GROUP

XWORDS = NB_GENES // 4
TBL = NB_GENES * NB_CAT
TBL_PAD = 912
EMB_WORDS = TBL * HIDDEN
EBUF_WORDS = TBL_PAD * HIDDEN
RSTRIDE = 920


def _sc_stage(emb_flat, cwb, xt):
    mesh = plsc.VectorSubcoreMesh(core_axis_name="c", subcore_axis_name="s")
    cp = pltpu.CompilerParams()
    if "needs_layout_passes" in pltpu.CompilerParams.__dataclass_fields__:
        cp = dataclasses.replace(cp, needs_layout_passes=False)

    @functools.partial(
        pl.kernel,
        compiler_params=cp,
        out_type=jax.ShapeDtypeStruct((BATCH, NB_CHROM), jnp.float32),
        mesh=mesh,
        scratch_types=[
            pltpu.VMEM((EBUF_WORDS,), jnp.float32),
            pltpu.VMEM((5 * LANES,), jnp.float32),
            pltpu.VMEM((TBL_PAD,), jnp.float32),
            pltpu.VMEM((GROUP * NB_GENES,), jnp.int32),
            pltpu.VMEM((GROUP, NB_CHROM), jnp.float32),
        ],
    )
    def body(emb_hbm, cwb_hbm, xt_hbm, out_hbm, ebuf, cwbuf, tbl,
             xbuf, obuf):
        wid = lax.axis_index("s") * NUM_CORES + lax.axis_index("c")
        base = wid * ROWS_PER_TILE

        pltpu.sync_copy(emb_hbm, ebuf.at[pl.ds(0, EMB_WORDS)])
        pltpu.sync_copy(cwb_hbm, cwbuf)

        lanes = jnp.arange(LANES, dtype=jnp.int32)
        cw0 = cwbuf[pl.ds(0 * LANES, LANES)]
        cw1 = cwbuf[pl.ds(1 * LANES, LANES)]
        cw2 = cwbuf[pl.ds(2 * LANES, LANES)]
        cw3 = cwbuf[pl.ds(3 * LANES, LANES)]
        cb = cwbuf[pl.ds(4 * LANES, LANES)]

        @pl.loop(0, TBL_PAD, step=LANES)
        def _(i):
            idx0 = (i + lanes) * HIDDEN
            v = (plsc.load_gather(ebuf, [idx0]) * cw0
                 + plsc.load_gather(ebuf, [idx0 + 1]) * cw1
                 + plsc.load_gather(ebuf, [idx0 + 2]) * cw2
                 + plsc.load_gather(ebuf, [idx0 + 3]) * cw3
                 + cb)
            tbl[pl.ds(i, LANES)] = v

        def compute(row0):
            for c in range(NB_CHROM):
                vals = []
                for j in range(GPC):
                    g = c * GPC + j
                    xv = xbuf[pl.ds(g * GROUP, LANES)]
                    tv = plsc.load_gather(tbl, [xv + 3 * g])
                    vals.append(tv)
                while len(vals) > 1:
                    vals = [a + b for a, b in zip(vals[::2], vals[1::2])] \
                        + ([vals[-1]] if len(vals) % 2 else [])
                cv = jnp.full((LANES,), c, jnp.int32)
                plsc.store_scatter(obuf, [lanes, cv], vals[0])
            pltpu.sync_copy(obuf, out_hbm.at[pl.ds(row0, GROUP), :])

        @pl.loop(0, GROUPS)
        def _(grp):
            gidx = wid * GROUPS + grp
            pltpu.sync_copy(xt_hbm.at[gidx], xbuf)
            compute(gidx * GROUP)

    return body(emb_flat, cwb, xt)


def _tc_tail(conv, lin_w, lin_b, bn_gamma, bn_beta):
    def body(a_ref, w_ref, b_ref, g_ref, bb_ref, o_ref):
        a = jnp.maximum(a_ref[...], 0.0)
        z = jnp.dot(a, w_ref[...],
                    preferred_element_type=jnp.float32)
        z = z + b_ref[...]
        mean = jnp.mean(z, axis=0, keepdims=True)
        zc = z - mean
        var = jnp.mean(zc * zc, axis=0, keepdims=True)
        inv = lax.rsqrt(var + 1e-5)
        o_ref[...] = zc * (inv * g_ref[...]) + bb_ref[...]

    return pl.pallas_call(
        body,
        out_shape=jax.ShapeDtypeStruct((BATCH, SIG), jnp.float32),
    )(conv, lin_w, lin_b, bn_gamma, bn_beta)


def kernel(x, emb, conv_w, conv_b, lin_w, lin_b, bn_gamma, bn_beta):
    emb_flat = emb.reshape(-1)
    cw = jnp.concatenate([conv_w, conv_b.reshape(1)]) * np.float32(1.0 / GPC)
    cwb = jnp.broadcast_to(cw[:, None], (5, LANES)).reshape(-1)
    xt = x.reshape(BATCH // GROUP, GROUP, NB_GENES)
    xt = xt.transpose(0, 2, 1).reshape(BATCH // GROUP, NB_GENES * GROUP)
    conv = _sc_stage(emb_flat, cwb, xt)
    return _tc_tail(conv, lin_w,
                    lin_b.reshape(1, SIG),
                    bn_gamma.reshape(1, SIG),
                    bn_beta.reshape(1, SIG))


# speedup vs baseline: 1.7233x; 1.0206x over previous
import dataclasses
import functools

import jax
import jax.numpy as jnp
import numpy as np
from jax import lax
from jax.experimental import pallas as pl
from jax.experimental.pallas import tpu as pltpu
from jax.experimental.pallas import tpu_sc as plsc

NB_GENES = 300
NB_CHROM = 10
GPC = 30
HIDDEN = 4
SIG = 16
BATCH = 16384
NB_CAT = 3

NUM_CORES = 2
NUM_SUBCORES = 16
NUM_TILES = NUM_CORES * NUM_SUBCORES
LANES = 16

ROWS_PER_TILE = BATCH // NUM_TILES
GROUP = 16
GROUPS = ROWS_PER_TILE // GROUP

XWORDS = NB_GENES // 4
TBL = NB_GENES * NB_CAT
TBL_PAD = 912
EMB_WORDS = TBL * HIDDEN
EBUF_WORDS = TBL_PAD * HIDDEN
RSTRIDE = 920


def _sc_stage(emb_flat, cwb, x):
    mesh = plsc.VectorSubcoreMesh(core_axis_name="c", subcore_axis_name="s")
    cp = pltpu.CompilerParams()
    if "needs_layout_passes" in pltpu.CompilerParams.__dataclass_fields__:
        cp = dataclasses.replace(cp, needs_layout_passes=False)

    @functools.partial(
        pl.kernel,
        compiler_params=cp,
        out_type=jax.ShapeDtypeStruct((BATCH, NB_CHROM), jnp.float32),
        mesh=mesh,
        scratch_types=[
            pltpu.VMEM((EBUF_WORDS,), jnp.float32),
            pltpu.VMEM((5 * LANES,), jnp.float32),
            pltpu.VMEM((LANES * RSTRIDE,), jnp.float32),
            pltpu.VMEM((GROUP, NB_GENES), jnp.int32),
            pltpu.VMEM((GROUP, NB_GENES), jnp.int32),
            pltpu.VMEM((GROUP, NB_CHROM), jnp.float32),
            pltpu.SemaphoreType.DMA,
            pltpu.SemaphoreType.DMA,
        ],
    )
    def body(emb_hbm, cwb_hbm, x_hbm, out_hbm, ebuf, cwbuf, tbl,
             xbuf0, xbuf1, obuf, sem0, sem1):
        wid = lax.axis_index("s") * NUM_CORES + lax.axis_index("c")
        base = wid * ROWS_PER_TILE

        pltpu.sync_copy(emb_hbm, ebuf.at[pl.ds(0, EMB_WORDS)])
        pltpu.sync_copy(cwb_hbm, cwbuf)

        lanes = jnp.arange(LANES, dtype=jnp.int32)
        cw0 = cwbuf[pl.ds(0 * LANES, LANES)]
        cw1 = cwbuf[pl.ds(1 * LANES, LANES)]
        cw2 = cwbuf[pl.ds(2 * LANES, LANES)]
        cw3 = cwbuf[pl.ds(3 * LANES, LANES)]
        cb = cwbuf[pl.ds(4 * LANES, LANES)]

        @pl.loop(0, TBL_PAD, step=LANES)
        def _(i):
            idx0 = (i + lanes) * HIDDEN
            v = (plsc.load_gather(ebuf, [idx0]) * cw0
                 + plsc.load_gather(ebuf, [idx0 + 1]) * cw1
                 + plsc.load_gather(ebuf, [idx0 + 2]) * cw2
                 + plsc.load_gather(ebuf, [idx0 + 3]) * cw3
                 + cb)
            for r in range(LANES):
                tbl[pl.ds(r * RSTRIDE + i, LANES)] = v

        tbase = lanes * RSTRIDE

        def compute(row0, xb):
            @pl.loop(0, NB_CHROM)
            def _(c):
                g0 = c * GPC
                vals = []
                for j in range(GPC):
                    g = g0 + j
                    gv = jnp.full((LANES,), g, jnp.int32)
                    xv = plsc.load_gather(xb, [lanes, gv])
                    vals.append(plsc.load_gather(tbl, [tbase + (xv + 3 * g)]))
                while len(vals) > 1:
                    vals = [a + b for a, b in zip(vals[::2], vals[1::2])] \
                        + ([vals[-1]] if len(vals) % 2 else [])
                cv = jnp.full((LANES,), c, jnp.int32)
                plsc.store_scatter(obuf, [lanes, cv], vals[0])
            pltpu.sync_copy(obuf, out_hbm.at[pl.ds(row0, GROUP), :])

        pltpu.async_copy(x_hbm.at[pl.ds(base, GROUP), :], xbuf0, sem0)
        pltpu.async_copy(x_hbm.at[pl.ds(base + GROUP, GROUP), :], xbuf1, sem1)

        @pl.loop(0, GROUPS, step=2)
        def _(grp):
            row0 = base + grp * GROUP
            pltpu.make_async_copy(x_hbm.at[pl.ds(0, GROUP), :],
                                  xbuf0, sem0).wait()
            compute(row0, xbuf0)

            @pl.when(grp + 2 < GROUPS)
            def _():
                pltpu.async_copy(
                    x_hbm.at[pl.ds(row0 + 2 * GROUP, GROUP), :], xbuf0, sem0)

            pltpu.make_async_copy(x_hbm.at[pl.ds(0, GROUP), :],
                                  xbuf1, sem1).wait()
            compute(row0 + GROUP, xbuf1)

            @pl.when(grp + 3 < GROUPS)
            def _():
                pltpu.async_copy(
                    x_hbm.at[pl.ds(row0 + 3 * GROUP, GROUP), :], xbuf1, sem1)

    return body(emb_flat, cwb, x)


def _tc_tail(conv, lin_w, lin_b, bn_gamma, bn_beta):
    def body(a_ref, w_ref, b_ref, g_ref, bb_ref, o_ref):
        a = jnp.maximum(a_ref[...], 0.0)
        z = jnp.dot(a, w_ref[...],
                    preferred_element_type=jnp.float32)
        z = z + b_ref[...]
        mean = jnp.mean(z, axis=0, keepdims=True)
        zc = z - mean
        var = jnp.mean(zc * zc, axis=0, keepdims=True)
        inv = lax.rsqrt(var + 1e-5)
        o_ref[...] = zc * (inv * g_ref[...]) + bb_ref[...]

    return pl.pallas_call(
        body,
        out_shape=jax.ShapeDtypeStruct((BATCH, SIG), jnp.float32),
    )(conv, lin_w, lin_b, bn_gamma, bn_beta)


def kernel(x, emb, conv_w, conv_b, lin_w, lin_b, bn_gamma, bn_beta):
    emb_flat = emb.reshape(-1)
    cw = jnp.concatenate([conv_w, conv_b.reshape(1)]) * np.float32(1.0 / GPC)
    cwb = jnp.broadcast_to(cw[:, None], (5, LANES)).reshape(-1)
    conv = _sc_stage(emb_flat, cwb, x)
    return _tc_tail(conv, lin_w,
                    lin_b.reshape(1, SIG),
                    bn_gamma.reshape(1, SIG),
                    bn_beta.reshape(1, SIG))


# speedup vs baseline: 1.7586x; 1.0205x over previous
import dataclasses
import functools

import jax
import jax.numpy as jnp
import numpy as np
from jax import lax
from jax.experimental import pallas as pl
from jax.experimental.pallas import tpu as pltpu
from jax.experimental.pallas import tpu_sc as plsc

NB_GENES = 300
NB_CHROM = 10
GPC = 30
HIDDEN = 4
SIG = 16
BATCH = 16384
NB_CAT = 3

NUM_CORES = 2
NUM_SUBCORES = 16
NUM_TILES = NUM_CORES * NUM_SUBCORES
LANES = 16

ROWS_PER_TILE = BATCH // NUM_TILES
GROUP = 16
GROUPS = ROWS_PER_TILE // GROUP

XWORDS = NB_GENES // 4
TBL = NB_GENES * NB_CAT
TBL_PAD = 912
EMB_WORDS = TBL * HIDDEN
EBUF_WORDS = TBL_PAD * HIDDEN
RSTRIDE = 920


def _sc_stage(emb_flat, cwb, x):
    mesh = plsc.VectorSubcoreMesh(core_axis_name="c", subcore_axis_name="s")
    cp = pltpu.CompilerParams()
    if "needs_layout_passes" in pltpu.CompilerParams.__dataclass_fields__:
        cp = dataclasses.replace(cp, needs_layout_passes=False)

    @functools.partial(
        pl.kernel,
        compiler_params=cp,
        out_type=jax.ShapeDtypeStruct((BATCH, NB_CHROM), jnp.float32),
        mesh=mesh,
        scratch_types=[
            pltpu.VMEM((EBUF_WORDS,), jnp.float32),
            pltpu.VMEM((5 * LANES,), jnp.float32),
            pltpu.VMEM((TBL_PAD,), jnp.float32),
            pltpu.VMEM((GROUP, NB_GENES), jnp.int32),
            pltpu.VMEM((GROUP, NB_GENES), jnp.int32),
            pltpu.VMEM((GROUP, NB_CHROM), jnp.float32),
            pltpu.SemaphoreType.DMA,
            pltpu.SemaphoreType.DMA,
        ],
    )
    def body(emb_hbm, cwb_hbm, x_hbm, out_hbm, ebuf, cwbuf, tbl,
             xbuf0, xbuf1, obuf, sem0, sem1):
        wid = lax.axis_index("s") * NUM_CORES + lax.axis_index("c")
        base = wid * ROWS_PER_TILE

        pltpu.sync_copy(emb_hbm, ebuf.at[pl.ds(0, EMB_WORDS)])
        pltpu.sync_copy(cwb_hbm, cwbuf)

        lanes = jnp.arange(LANES, dtype=jnp.int32)
        cw0 = cwbuf[pl.ds(0 * LANES, LANES)]
        cw1 = cwbuf[pl.ds(1 * LANES, LANES)]
        cw2 = cwbuf[pl.ds(2 * LANES, LANES)]
        cw3 = cwbuf[pl.ds(3 * LANES, LANES)]
        cb = cwbuf[pl.ds(4 * LANES, LANES)]

        @pl.loop(0, TBL_PAD, step=LANES)
        def _(i):
            idx0 = (i + lanes) * HIDDEN
            v = (plsc.load_gather(ebuf, [idx0]) * cw0
                 + plsc.load_gather(ebuf, [idx0 + 1]) * cw1
                 + plsc.load_gather(ebuf, [idx0 + 2]) * cw2
                 + plsc.load_gather(ebuf, [idx0 + 3]) * cw3
                 + cb)
            tbl[pl.ds(i, LANES)] = v

        def compute(row0, xb):
            @pl.loop(0, NB_CHROM)
            def _(c):
                g0 = c * GPC
                vals = []
                for j in range(GPC):
                    g = g0 + j
                    gv = jnp.full((LANES,), g, jnp.int32)
                    xv = plsc.load_gather(xb, [lanes, gv])
                    vals.append(plsc.load_gather(tbl, [xv + 3 * g]))
                while len(vals) > 1:
                    vals = [a + b for a, b in zip(vals[::2], vals[1::2])] \
                        + ([vals[-1]] if len(vals) % 2 else [])
                cv = jnp.full((LANES,), c, jnp.int32)
                plsc.store_scatter(obuf, [lanes, cv], vals[0])
            pltpu.sync_copy(obuf, out_hbm.at[pl.ds(row0, GROUP), :])

        pltpu.async_copy(x_hbm.at[pl.ds(base, GROUP), :], xbuf0, sem0)
        pltpu.async_copy(x_hbm.at[pl.ds(base + GROUP, GROUP), :], xbuf1, sem1)

        @pl.loop(0, GROUPS, step=2)
        def _(grp):
            row0 = base + grp * GROUP
            pltpu.make_async_copy(x_hbm.at[pl.ds(0, GROUP), :],
                                  xbuf0, sem0).wait()
            compute(row0, xbuf0)

            @pl.when(grp + 2 < GROUPS)
            def _():
                pltpu.async_copy(
                    x_hbm.at[pl.ds(row0 + 2 * GROUP, GROUP), :], xbuf0, sem0)

            pltpu.make_async_copy(x_hbm.at[pl.ds(0, GROUP), :],
                                  xbuf1, sem1).wait()
            compute(row0 + GROUP, xbuf1)

            @pl.when(grp + 3 < GROUPS)
            def _():
                pltpu.async_copy(
                    x_hbm.at[pl.ds(row0 + 3 * GROUP, GROUP), :], xbuf1, sem1)

    return body(emb_flat, cwb, x)


def _tc_tail(conv, lin_w, lin_b, bn_gamma, bn_beta):
    def body(a_ref, w_ref, b_ref, g_ref, bb_ref, o_ref):
        a = jnp.maximum(a_ref[...], 0.0)
        z = jnp.dot(a, w_ref[...],
                    preferred_element_type=jnp.float32)
        z = z + b_ref[...]
        mean = jnp.mean(z, axis=0, keepdims=True)
        zc = z - mean
        var = jnp.mean(zc * zc, axis=0, keepdims=True)
        inv = lax.rsqrt(var + 1e-5)
        o_ref[...] = zc * (inv * g_ref[...]) + bb_ref[...]

    return pl.pallas_call(
        body,
        out_shape=jax.ShapeDtypeStruct((BATCH, SIG), jnp.float32),
    )(conv, lin_w, lin_b, bn_gamma, bn_beta)


def kernel(x, emb, conv_w, conv_b, lin_w, lin_b, bn_gamma, bn_beta):
    emb_flat = emb.reshape(-1)
    cw = jnp.concatenate([conv_w, conv_b.reshape(1)]) * np.float32(1.0 / GPC)
    cwb = jnp.broadcast_to(cw[:, None], (5, LANES)).reshape(-1)
    conv = _sc_stage(emb_flat, cwb, x)
    return _tc_tail(conv, lin_w,
                    lin_b.reshape(1, SIG),
                    bn_gamma.reshape(1, SIG),
                    bn_beta.reshape(1, SIG))
